# baseline (speedup 1.0000x reference)
import dataclasses
import functools


import jax
import jax.numpy as jnp
from jax import lax
from jax.experimental import pallas as pl
from jax.experimental.pallas import tpu as pltpu
from jax.experimental.pallas import tpu_sc as plsc

N_SRC = 10000
N_DST = 10000
E_EDGES = 320000
D = 128
OUT = 128

NUM_CORES = 2
NUM_SUBCORES = 16
NUM_TILES = NUM_CORES * NUM_SUBCORES
EDGES_PER_TILE = E_EDGES // NUM_TILES
CHUNK = 120
NCHUNKS = EDGES_PER_TILE // CHUNK
TAIL = EDGES_PER_TILE - NCHUNKS * CHUNK
NBUF = 3
N_PAD = 10112
ROWS_PER_TILE = N_PAD // NUM_SUBCORES
LANES = 16
GROUPS = EDGES_PER_TILE // LANES


def _tables_body(nfs_ref, nfd_ref, sw_ref, q_ref, degs_ref, degd_ref, out_ref):
    w = sw_ref[...]
    hu = jnp.sum(nfs_ref[...] * w[:, 0][None, :], axis=1)
    hv = jnp.sum(nfd_ref[...] * w[:, 1][None, :], axis=1)
    coef = lax.rsqrt(degs_ref[...].astype(jnp.float32) + 1.0) / (
        q_ref[...] * float(E_EDGES))
    norm_dst = lax.rsqrt(degd_ref[...].astype(jnp.float32) + 1.0)
    out_ref[0, :] = coef
    out_ref[1, :] = hu
    out_ref[2, :] = norm_dst
    out_ref[3, :] = hv


def _attn_kernel_body(src_hbm, dst_hbm, tabs_hbm, attn_hbm,
                      coef_ref, hu_ref, nd_ref, hv_ref,
                      sidx_ref, didx_ref, attn_ref, sem):
    c = lax.axis_index("c")
    s = lax.axis_index("s")
    base_edge = (c * NUM_SUBCORES + s) * EDGES_PER_TILE

    copies = [
        (tabs_hbm.at[pl.ds(0 * N_SRC, N_SRC)], coef_ref),
        (tabs_hbm.at[pl.ds(1 * N_SRC, N_SRC)], hu_ref),
        (tabs_hbm.at[pl.ds(2 * N_SRC, N_SRC)], nd_ref),
        (tabs_hbm.at[pl.ds(3 * N_SRC, N_SRC)], hv_ref),
        (src_hbm.at[pl.ds(base_edge, EDGES_PER_TILE)], sidx_ref),
        (dst_hbm.at[pl.ds(base_edge, EDGES_PER_TILE)], didx_ref),
    ]
    for src_, dst_ in copies:
        pltpu.async_copy(src_, dst_, sem)
    for src_, dst_ in copies:
        pltpu.make_async_copy(src_, dst_, sem).wait()

    @pl.loop(0, GROUPS, step=5)
    def _(g):
        for gg in range(5):
            sl = pl.ds((g + gg) * LANES, LANES)
            sv = sidx_ref[sl]
            dv = didx_ref[sl]
            cs = plsc.load_gather(coef_ref, [sv])
            hus = plsc.load_gather(hu_ref, [sv])
            nd = plsc.load_gather(nd_ref, [dv])
            hvs = plsc.load_gather(hv_ref, [dv])
            attn_ref[sl] = cs * nd * (jnp.maximum(hus + hvs, 0.0) + 0.1)

    pltpu.sync_copy(attn_ref, attn_hbm.at[pl.ds(base_edge, EDGES_PER_TILE)])


def _agg_kernel_body(src_hbm, dst_hbm, hidden_hbm, attn_hbm, zeros_hbm,
                     out_hbm,
                     sidx_refs, didx_refs, attn_refs, rows_refs,
                     sidx_t, didx_t, attn_t,
                     pf_sems, g_sems, sc_sems, z_sem, acc_ref):
    c = lax.axis_index("c")
    s = lax.axis_index("s")
    base_edge = (c * NUM_SUBCORES + s) * EDGES_PER_TILE
    row0 = s * ROWS_PER_TILE

    pltpu.async_copy(zeros_hbm, acc_ref.at[pl.ds(row0, ROWS_PER_TILE)], z_sem)

    def start_pf(j, b):
        base = base_edge + j * CHUNK
        pltpu.async_copy(src_hbm.at[pl.ds(base, CHUNK)], sidx_refs[b],
                         pf_sems[b])
        pltpu.async_copy(dst_hbm.at[pl.ds(base, CHUNK)], didx_refs[b],
                         pf_sems[b])
        pltpu.async_copy(attn_hbm.at[pl.ds(base, CHUNK)], attn_refs[b],
                         pf_sems[b])

    def wait_pf(b):
        pltpu.make_async_copy(src_hbm.at[pl.ds(0, CHUNK)], sidx_refs[b],
                              pf_sems[b]).wait()
        pltpu.make_async_copy(dst_hbm.at[pl.ds(0, CHUNK)], didx_refs[b],
                              pf_sems[b]).wait()
        pltpu.make_async_copy(attn_hbm.at[pl.ds(0, CHUNK)], attn_refs[b],
                              pf_sems[b]).wait()

    def start_gather(j, b):
        del j
        pltpu.async_copy(hidden_hbm.at[sidx_refs[b]], rows_refs[b], g_sems[b])

    def wait_gather(b):
        pltpu.make_async_copy(hidden_hbm.at[sidx_refs[b]], rows_refs[b],
                              g_sems[b]).wait()

    def start_scatter(b):
        pltpu.async_copy(rows_refs[b], acc_ref.at[didx_refs[b]], sc_sems[b],
                         add=True)

    def wait_scatter(b):
        pltpu.make_async_copy(rows_refs[b], acc_ref.at[didx_refs[b]],
                              sc_sems[b]).wait()

    def scale(b):
        rows = rows_refs[b]
        attn = attn_refs[b]

        @pl.loop(0, CHUNK, step=2)
        def _(e):
            a0 = plsc.load_gather(attn, [jnp.full((LANES,), e, jnp.int32)])
            a1 = plsc.load_gather(attn, [jnp.full((LANES,), e + 1, jnp.int32)])
            for g in range(D // LANES):
                sl = pl.ds(g * LANES, LANES)
                rows[e, sl] = rows[e, sl] * a0
                rows[e + 1, sl] = rows[e + 1, sl] * a1

    for b in range(NBUF):
        start_pf(b, b)
    for b in range(NBUF):
        wait_pf(b)
        start_gather(b, b)

    pltpu.make_async_copy(zeros_hbm, acc_ref.at[pl.ds(row0, ROWS_PER_TILE)],
                          z_sem).wait()
    plsc.subcore_barrier()

    steady = (NCHUNKS - NBUF) // NBUF

    @pl.loop(0, steady)
    def _(k):
        j = k * NBUF
        for b in range(NBUF):
            wait_gather(b)
            scale(b)
            start_scatter(b)
        for b in range(NBUF):
            wait_scatter(b)
            start_pf(j + NBUF + b, b)
            wait_pf(b)
            start_gather(j + NBUF + b, b)

    for b in range(NBUF):
        wait_gather(b)
        scale(b)
        start_scatter(b)
    for i, j in enumerate(range(NBUF * (steady + 1), NCHUNKS)):
        b = i
        wait_scatter(b)
        start_pf(j, b)
        wait_pf(b)
        start_gather(j, b)
    for i in range(NCHUNKS - NBUF * (steady + 1)):
        wait_gather(i)
        scale(i)
        start_scatter(i)
    for b in range(NBUF):
        wait_scatter(b)

    tbase = base_edge + NCHUNKS * CHUNK
    pltpu.sync_copy(src_hbm.at[pl.ds(tbase, TAIL)], sidx_t)
    pltpu.sync_copy(dst_hbm.at[pl.ds(tbase, TAIL)], didx_t)
    pltpu.sync_copy(attn_hbm.at[pl.ds(tbase, TAIL)], attn_t)
    trows = rows_refs[0].at[pl.ds(0, TAIL)]
    pltpu.sync_copy(hidden_hbm.at[sidx_t], trows)

    @pl.loop(0, TAIL)
    def _(e):
        a = plsc.load_gather(attn_t, [jnp.full((LANES,), e, jnp.int32)])
        for g in range(D // LANES):
            sl = pl.ds(g * LANES, LANES)
            rows_refs[0][e, sl] = rows_refs[0][e, sl] * a

    pltpu.sync_copy(trows, acc_ref.at[didx_t], add=True)

    plsc.subcore_barrier()
    pltpu.sync_copy(acc_ref.at[pl.ds(row0, ROWS_PER_TILE)],
                    out_hbm.at[c, pl.ds(row0, ROWS_PER_TILE)])


def _final_body(part_ref, w_ref, b_ref, out_ref):
    h = part_ref[0, :N_DST, :] + part_ref[1, :N_DST, :]
    rst = jax.lax.dot_general(
        h, w_ref[...],
        dimension_numbers=(((1,), (1,)), ((), ())),
        precision=lax.Precision.HIGHEST,
        preferred_element_type=jnp.float32)
    out_ref[...] = rst + b_ref[...][None, :]


@jax.jit
def kernel(hidden_feat, node_feat_src, node_feat_dst, sample_weights, q_probs,
           W_neigh, b_neigh, edge_index, deg_src, deg_dst):
    tabs = pl.pallas_call(
        _tables_body,
        out_shape=jax.ShapeDtypeStruct((4, N_SRC), jnp.float32),
    )(node_feat_src, node_feat_dst, sample_weights, q_probs, deg_src, deg_dst)
    tabs = tabs.reshape(4 * N_SRC)

    src = edge_index[0]
    dst = edge_index[1]
    zeros_rows = jnp.zeros((ROWS_PER_TILE, D), jnp.float32)

    mesh = plsc.VectorSubcoreMesh(core_axis_name="c", subcore_axis_name="s")
    sc_params = pltpu.CompilerParams()
    if "needs_layout_passes" in pltpu.CompilerParams.__dataclass_fields__:
        sc_params = dataclasses.replace(sc_params, needs_layout_passes=False)

    attn_kernel = functools.partial(
        pl.kernel,
        compiler_params=sc_params,
        out_type=jax.ShapeDtypeStruct((E_EDGES,), jnp.float32),
        mesh=mesh,
        scratch_types=[
            pltpu.VMEM((N_SRC,), jnp.float32),
            pltpu.VMEM((N_SRC,), jnp.float32),
            pltpu.VMEM((N_DST,), jnp.float32),
            pltpu.VMEM((N_DST,), jnp.float32),
            pltpu.VMEM((EDGES_PER_TILE,), jnp.int32),
            pltpu.VMEM((EDGES_PER_TILE,), jnp.int32),
            pltpu.VMEM((EDGES_PER_TILE,), jnp.float32),
            pltpu.SemaphoreType.DMA,
        ],
    )(_attn_kernel_body)
    attn_all = attn_kernel(src, dst, tabs)

    agg_kernel = functools.partial(
        pl.kernel,
        compiler_params=sc_params,
        out_type=jax.ShapeDtypeStruct((NUM_CORES, N_PAD, D), jnp.float32),
        mesh=mesh,
        scratch_types=[
            [pltpu.VMEM((CHUNK,), jnp.int32) for _ in range(NBUF)],
            [pltpu.VMEM((CHUNK,), jnp.int32) for _ in range(NBUF)],
            [pltpu.VMEM((CHUNK,), jnp.float32) for _ in range(NBUF)],
            [pltpu.VMEM((CHUNK, D), jnp.float32) for _ in range(NBUF)],
            pltpu.VMEM((TAIL,), jnp.int32),
            pltpu.VMEM((TAIL,), jnp.int32),
            pltpu.VMEM((TAIL,), jnp.float32),
            [pltpu.SemaphoreType.DMA for _ in range(NBUF)],
            [pltpu.SemaphoreType.DMA for _ in range(NBUF)],
            [pltpu.SemaphoreType.DMA for _ in range(NBUF)],
            pltpu.SemaphoreType.DMA,
            pltpu.VMEM_SHARED((N_PAD, D), jnp.float32),
        ],
    )(_agg_kernel_body)
    partials = agg_kernel(src, dst, hidden_feat, attn_all, zeros_rows)

    rst = pl.pallas_call(
        _final_body,
        out_shape=jax.ShapeDtypeStruct((N_DST, OUT), jnp.float32),
    )(partials, W_neigh, b_neigh)
    return rst


# speedup vs baseline: 1.0487x; 1.0487x over previous
import dataclasses
import functools


import jax
import jax.numpy as jnp
from jax import lax
from jax.experimental import pallas as pl
from jax.experimental.pallas import tpu as pltpu
from jax.experimental.pallas import tpu_sc as plsc

N_SRC = 10000
N_DST = 10000
E_EDGES = 320000
D = 128
OUT = 128

NUM_CORES = 2
NUM_SUBCORES = 16
NUM_TILES = NUM_CORES * NUM_SUBCORES
EDGES_PER_TILE = E_EDGES // NUM_TILES
CHUNK = 120
NCHUNKS = EDGES_PER_TILE // CHUNK
TAIL = EDGES_PER_TILE - NCHUNKS * CHUNK
NBUF = 3
N_PAD = 10112
ROWS_PER_TILE = N_PAD // NUM_SUBCORES
LANES = 16
GROUPS = EDGES_PER_TILE // LANES


def _tables_body(nfs_ref, nfd_ref, sw_ref, q_ref, degs_ref, degd_ref, out_ref):
    w = sw_ref[...]
    hu = jnp.sum(nfs_ref[...] * w[:, 0][None, :], axis=1)
    hv = jnp.sum(nfd_ref[...] * w[:, 1][None, :], axis=1)
    coef = lax.rsqrt(degs_ref[...].astype(jnp.float32) + 1.0) / (
        q_ref[...] * float(E_EDGES))
    norm_dst = lax.rsqrt(degd_ref[...].astype(jnp.float32) + 1.0)
    out_ref[pl.ds(0 * N_SRC, N_SRC)] = coef
    out_ref[pl.ds(1 * N_SRC, N_SRC)] = hu
    out_ref[pl.ds(2 * N_SRC, N_SRC)] = norm_dst
    out_ref[pl.ds(3 * N_SRC, N_SRC)] = hv


def _attn_kernel_body(eidx_hbm, tabs_hbm, attn_hbm,
                      coef_ref, hu_ref, nd_ref, hv_ref,
                      sidx_ref, didx_ref, attn_ref, sem):
    c = lax.axis_index("c")
    s = lax.axis_index("s")
    base_edge = (c * NUM_SUBCORES + s) * EDGES_PER_TILE

    copies = [
        (tabs_hbm.at[pl.ds(0 * N_SRC, N_SRC)], coef_ref),
        (tabs_hbm.at[pl.ds(1 * N_SRC, N_SRC)], hu_ref),
        (tabs_hbm.at[pl.ds(2 * N_SRC, N_SRC)], nd_ref),
        (tabs_hbm.at[pl.ds(3 * N_SRC, N_SRC)], hv_ref),
        (eidx_hbm.at[pl.ds(base_edge, EDGES_PER_TILE)], sidx_ref),
        (eidx_hbm.at[pl.ds(E_EDGES + base_edge, EDGES_PER_TILE)], didx_ref),
    ]
    for src_, dst_ in copies:
        pltpu.async_copy(src_, dst_, sem)
    for src_, dst_ in copies:
        pltpu.make_async_copy(src_, dst_, sem).wait()

    @pl.loop(0, GROUPS, step=5)
    def _(g):
        for gg in range(5):
            sl = pl.ds((g + gg) * LANES, LANES)
            sv = sidx_ref[sl]
            dv = didx_ref[sl]
            cs = plsc.load_gather(coef_ref, [sv])
            hus = plsc.load_gather(hu_ref, [sv])
            nd = plsc.load_gather(nd_ref, [dv])
            hvs = plsc.load_gather(hv_ref, [dv])
            attn_ref[sl] = cs * nd * (jnp.maximum(hus + hvs, 0.0) + 0.1)

    pltpu.sync_copy(attn_ref, attn_hbm.at[pl.ds(base_edge, EDGES_PER_TILE)])


def _agg_kernel_body(eidx_hbm, hidden_hbm, attn_hbm, zeros_hbm,
                     out_hbm,
                     sidx_refs, didx_refs, attn_refs, rows_refs,
                     sidx_t, didx_t, attn_t,
                     pf_sems, g_sems, sc_sems, z_sem, acc_ref):
    c = lax.axis_index("c")
    s = lax.axis_index("s")
    base_edge = (c * NUM_SUBCORES + s) * EDGES_PER_TILE
    row0 = s * ROWS_PER_TILE

    pltpu.async_copy(zeros_hbm, acc_ref.at[pl.ds(row0, ROWS_PER_TILE)], z_sem)

    def start_pf(j, b):
        base = base_edge + j * CHUNK
        pltpu.async_copy(eidx_hbm.at[pl.ds(base, CHUNK)], sidx_refs[b],
                         pf_sems[b])
        pltpu.async_copy(eidx_hbm.at[pl.ds(E_EDGES + base, CHUNK)],
                         didx_refs[b], pf_sems[b])
        pltpu.async_copy(attn_hbm.at[pl.ds(base, CHUNK)], attn_refs[b],
                         pf_sems[b])

    def wait_pf(b):
        pltpu.make_async_copy(eidx_hbm.at[pl.ds(0, CHUNK)], sidx_refs[b],
                              pf_sems[b]).wait()
        pltpu.make_async_copy(eidx_hbm.at[pl.ds(0, CHUNK)], didx_refs[b],
                              pf_sems[b]).wait()
        pltpu.make_async_copy(attn_hbm.at[pl.ds(0, CHUNK)], attn_refs[b],
                              pf_sems[b]).wait()

    def start_gather(j, b):
        del j
        pltpu.async_copy(hidden_hbm.at[sidx_refs[b]], rows_refs[b], g_sems[b])

    def wait_gather(b):
        pltpu.make_async_copy(hidden_hbm.at[sidx_refs[b]], rows_refs[b],
                              g_sems[b]).wait()

    def start_scatter(b):
        pltpu.async_copy(rows_refs[b], acc_ref.at[didx_refs[b]], sc_sems[b],
                         add=True)

    def wait_scatter(b):
        pltpu.make_async_copy(rows_refs[b], acc_ref.at[didx_refs[b]],
                              sc_sems[b]).wait()

    def scale(b):
        rows = rows_refs[b]
        attn = attn_refs[b]

        @pl.loop(0, CHUNK, step=2)
        def _(e):
            a0 = plsc.load_gather(attn, [jnp.full((LANES,), e, jnp.int32)])
            a1 = plsc.load_gather(attn, [jnp.full((LANES,), e + 1, jnp.int32)])
            for g in range(D // LANES):
                sl = pl.ds(g * LANES, LANES)
                rows[e, sl] = rows[e, sl] * a0
                rows[e + 1, sl] = rows[e + 1, sl] * a1

    for b in range(NBUF):
        start_pf(b, b)
    for b in range(NBUF):
        wait_pf(b)
        start_gather(b, b)

    pltpu.make_async_copy(zeros_hbm, acc_ref.at[pl.ds(row0, ROWS_PER_TILE)],
                          z_sem).wait()
    plsc.subcore_barrier()

    steady = (NCHUNKS - NBUF) // NBUF

    @pl.loop(0, steady)
    def _(k):
        j = k * NBUF
        for b in range(NBUF):
            wait_gather(b)
            scale(b)
            start_scatter(b)
        for b in range(NBUF):
            wait_scatter(b)
            start_pf(j + NBUF + b, b)
            wait_pf(b)
            start_gather(j + NBUF + b, b)

    for b in range(NBUF):
        wait_gather(b)
        scale(b)
        start_scatter(b)
    for i, j in enumerate(range(NBUF * (steady + 1), NCHUNKS)):
        b = i
        wait_scatter(b)
        start_pf(j, b)
        wait_pf(b)
        start_gather(j, b)
    for i in range(NCHUNKS - NBUF * (steady + 1)):
        wait_gather(i)
        scale(i)
        start_scatter(i)
    for b in range(NBUF):
        wait_scatter(b)

    tbase = base_edge + NCHUNKS * CHUNK
    pltpu.sync_copy(eidx_hbm.at[pl.ds(tbase, TAIL)], sidx_t)
    pltpu.sync_copy(eidx_hbm.at[pl.ds(E_EDGES + tbase, TAIL)], didx_t)
    pltpu.sync_copy(attn_hbm.at[pl.ds(tbase, TAIL)], attn_t)
    trows = rows_refs[0].at[pl.ds(0, TAIL)]
    pltpu.sync_copy(hidden_hbm.at[sidx_t], trows)

    @pl.loop(0, TAIL)
    def _(e):
        a = plsc.load_gather(attn_t, [jnp.full((LANES,), e, jnp.int32)])
        for g in range(D // LANES):
            sl = pl.ds(g * LANES, LANES)
            rows_refs[0][e, sl] = rows_refs[0][e, sl] * a

    pltpu.sync_copy(trows, acc_ref.at[didx_t], add=True)

    plsc.subcore_barrier()
    pltpu.sync_copy(acc_ref.at[pl.ds(row0, ROWS_PER_TILE)],
                    out_hbm.at[c, pl.ds(row0, ROWS_PER_TILE)])


def _final_body(part_ref, w_ref, b_ref, out_ref):
    h = part_ref[0, :N_DST, :] + part_ref[1, :N_DST, :]
    rst = jax.lax.dot_general(
        h, w_ref[...],
        dimension_numbers=(((1,), (1,)), ((), ())),
        precision=lax.Precision.HIGHEST,
        preferred_element_type=jnp.float32)
    out_ref[...] = rst + b_ref[...][None, :]


@jax.jit
def kernel(hidden_feat, node_feat_src, node_feat_dst, sample_weights, q_probs,
           W_neigh, b_neigh, edge_index, deg_src, deg_dst):
    tabs = pl.pallas_call(
        _tables_body,
        out_shape=jax.ShapeDtypeStruct((4 * N_SRC,), jnp.float32),
    )(node_feat_src, node_feat_dst, sample_weights, q_probs, deg_src, deg_dst)

    eidx = edge_index.reshape(2 * E_EDGES)
    zeros_rows = jnp.zeros((ROWS_PER_TILE, D), jnp.float32)

    mesh = plsc.VectorSubcoreMesh(core_axis_name="c", subcore_axis_name="s")
    sc_params = pltpu.CompilerParams()
    if "needs_layout_passes" in pltpu.CompilerParams.__dataclass_fields__:
        sc_params = dataclasses.replace(sc_params, needs_layout_passes=False)

    attn_kernel = functools.partial(
        pl.kernel,
        compiler_params=sc_params,
        out_type=jax.ShapeDtypeStruct((E_EDGES,), jnp.float32),
        mesh=mesh,
        scratch_types=[
            pltpu.VMEM((N_SRC,), jnp.float32),
            pltpu.VMEM((N_SRC,), jnp.float32),
            pltpu.VMEM((N_DST,), jnp.float32),
            pltpu.VMEM((N_DST,), jnp.float32),
            pltpu.VMEM((EDGES_PER_TILE,), jnp.int32),
            pltpu.VMEM((EDGES_PER_TILE,), jnp.int32),
            pltpu.VMEM((EDGES_PER_TILE,), jnp.float32),
            pltpu.SemaphoreType.DMA,
        ],
    )(_attn_kernel_body)
    attn_all = attn_kernel(eidx, tabs)

    agg_kernel = functools.partial(
        pl.kernel,
        compiler_params=sc_params,
        out_type=jax.ShapeDtypeStruct((NUM_CORES, N_PAD, D), jnp.float32),
        mesh=mesh,
        scratch_types=[
            [pltpu.VMEM((CHUNK,), jnp.int32) for _ in range(NBUF)],
            [pltpu.VMEM((CHUNK,), jnp.int32) for _ in range(NBUF)],
            [pltpu.VMEM((CHUNK,), jnp.float32) for _ in range(NBUF)],
            [pltpu.VMEM((CHUNK, D), jnp.float32) for _ in range(NBUF)],
            pltpu.VMEM((TAIL,), jnp.int32),
            pltpu.VMEM((TAIL,), jnp.int32),
            pltpu.VMEM((TAIL,), jnp.float32),
            [pltpu.SemaphoreType.DMA for _ in range(NBUF)],
            [pltpu.SemaphoreType.DMA for _ in range(NBUF)],
            [pltpu.SemaphoreType.DMA for _ in range(NBUF)],
            pltpu.SemaphoreType.DMA,
            pltpu.VMEM_SHARED((N_PAD, D), jnp.float32),
        ],
    )(_agg_kernel_body)
    partials = agg_kernel(eidx, hidden_feat, attn_all, zeros_rows)

    rst = pl.pallas_call(
        _final_body,
        out_shape=jax.ShapeDtypeStruct((N_DST, OUT), jnp.float32),
    )(partials, W_neigh, b_neigh)
    return rst


# speedup vs baseline: 1.0553x; 1.0063x over previous
import dataclasses
import functools


import jax
import jax.numpy as jnp
from jax import lax
from jax.experimental import pallas as pl
from jax.experimental.pallas import tpu as pltpu
from jax.experimental.pallas import tpu_sc as plsc

N_SRC = 10000
N_DST = 10000
E_EDGES = 320000
D = 128
OUT = 128

NUM_CORES = 2
NUM_SUBCORES = 16
NUM_TILES = NUM_CORES * NUM_SUBCORES
EDGES_PER_TILE = E_EDGES // NUM_TILES
CHUNK = 120
NCHUNKS = EDGES_PER_TILE // CHUNK
TAIL = EDGES_PER_TILE - NCHUNKS * CHUNK
NBUF = 3
N_PAD = 10112
ROWS_PER_TILE = N_PAD // NUM_SUBCORES
LANES = 16
GROUPS = EDGES_PER_TILE // LANES


def _tables_body(nfs_ref, nfd_ref, sw_ref, q_ref, degs_ref, degd_ref, out_ref):
    w = sw_ref[...]
    hu = jnp.sum(nfs_ref[...] * w[:, 0][None, :], axis=1)
    hv = jnp.sum(nfd_ref[...] * w[:, 1][None, :], axis=1)
    coef = lax.rsqrt(degs_ref[...].astype(jnp.float32) + 1.0) / (
        q_ref[...] * float(E_EDGES))
    norm_dst = lax.rsqrt(degd_ref[...].astype(jnp.float32) + 1.0)
    out_ref[pl.ds(0 * N_SRC, N_SRC)] = coef
    out_ref[pl.ds(1 * N_SRC, N_SRC)] = hu
    out_ref[pl.ds(2 * N_SRC, N_SRC)] = norm_dst
    out_ref[pl.ds(3 * N_SRC, N_SRC)] = hv


def _attn_kernel_body(eidx_hbm, tabs_hbm, attn_hbm,
                      coef_ref, hu_ref, nd_ref, hv_ref,
                      sidx_ref, didx_ref, attn_ref, sem):
    c = lax.axis_index("c")
    s = lax.axis_index("s")
    base_edge = (c * NUM_SUBCORES + s) * EDGES_PER_TILE

    copies = [
        (tabs_hbm.at[pl.ds(0 * N_SRC, N_SRC)], coef_ref),
        (tabs_hbm.at[pl.ds(1 * N_SRC, N_SRC)], hu_ref),
        (tabs_hbm.at[pl.ds(2 * N_SRC, N_SRC)], nd_ref),
        (tabs_hbm.at[pl.ds(3 * N_SRC, N_SRC)], hv_ref),
        (eidx_hbm.at[pl.ds(base_edge, EDGES_PER_TILE)], sidx_ref),
        (eidx_hbm.at[pl.ds(E_EDGES + base_edge, EDGES_PER_TILE)], didx_ref),
    ]
    for src_, dst_ in copies:
        pltpu.async_copy(src_, dst_, sem)
    for src_, dst_ in copies:
        pltpu.make_async_copy(src_, dst_, sem).wait()

    @pl.loop(0, GROUPS, step=5)
    def _(g):
        for gg in range(5):
            sl = pl.ds((g + gg) * LANES, LANES)
            sv = sidx_ref[sl]
            dv = didx_ref[sl]
            cs = plsc.load_gather(coef_ref, [sv])
            hus = plsc.load_gather(hu_ref, [sv])
            nd = plsc.load_gather(nd_ref, [dv])
            hvs = plsc.load_gather(hv_ref, [dv])
            attn_ref[sl] = cs * nd * (jnp.maximum(hus + hvs, 0.0) + 0.1)

    pltpu.sync_copy(attn_ref, attn_hbm.at[pl.ds(base_edge, EDGES_PER_TILE)])


def _agg_kernel_body(eidx_hbm, hidden_hbm, attn_hbm, zeros_hbm,
                     out_hbm,
                     sidx_refs, didx_refs, attn_refs, rows_refs,
                     sidx_t, didx_t, attn_t,
                     pf_sems, g_sems, sc_sems, z_sem, acc_ref):
    c = lax.axis_index("c")
    s = lax.axis_index("s")
    base_edge = (c * NUM_SUBCORES + s) * EDGES_PER_TILE
    row0 = s * ROWS_PER_TILE

    pltpu.async_copy(zeros_hbm, acc_ref.at[pl.ds(row0, ROWS_PER_TILE)], z_sem)

    def start_pf(j, b):
        base = base_edge + j * CHUNK
        pltpu.async_copy(eidx_hbm.at[pl.ds(base, CHUNK)], sidx_refs[b],
                         pf_sems[b])
        pltpu.async_copy(eidx_hbm.at[pl.ds(E_EDGES + base, CHUNK)],
                         didx_refs[b], pf_sems[b])
        pltpu.async_copy(attn_hbm.at[pl.ds(base, CHUNK)], attn_refs[b],
                         pf_sems[b])

    def wait_pf(b):
        pltpu.make_async_copy(eidx_hbm.at[pl.ds(0, CHUNK)], sidx_refs[b],
                              pf_sems[b]).wait()
        pltpu.make_async_copy(eidx_hbm.at[pl.ds(0, CHUNK)], didx_refs[b],
                              pf_sems[b]).wait()
        pltpu.make_async_copy(attn_hbm.at[pl.ds(0, CHUNK)], attn_refs[b],
                              pf_sems[b]).wait()

    def start_gather(j, b):
        del j
        pltpu.async_copy(hidden_hbm.at[sidx_refs[b]], rows_refs[b], g_sems[b])

    def wait_gather(b):
        pltpu.make_async_copy(hidden_hbm.at[sidx_refs[b]], rows_refs[b],
                              g_sems[b]).wait()

    def start_scatter(b):
        pltpu.async_copy(rows_refs[b], acc_ref.at[didx_refs[b]], sc_sems[b],
                         add=True)

    def wait_scatter(b):
        pltpu.make_async_copy(rows_refs[b], acc_ref.at[didx_refs[b]],
                              sc_sems[b]).wait()

    def scale(b):
        rows = rows_refs[b]
        attn = attn_refs[b]

        @pl.loop(0, CHUNK, step=4)
        def _(e):
            avs = [
                plsc.load_gather(attn, [jnp.full((LANES,), e + i, jnp.int32)])
                for i in range(4)
            ]
            for g in range(D // LANES):
                sl = pl.ds(g * LANES, LANES)
                for i in range(4):
                    rows[e + i, sl] = rows[e + i, sl] * avs[i]

    for b in range(NBUF):
        start_pf(b, b)
    for b in range(NBUF):
        wait_pf(b)
        start_gather(b, b)

    pltpu.make_async_copy(zeros_hbm, acc_ref.at[pl.ds(row0, ROWS_PER_TILE)],
                          z_sem).wait()
    plsc.subcore_barrier()

    steady = (NCHUNKS - NBUF) // NBUF

    @pl.loop(0, steady)
    def _(k):
        j = k * NBUF
        for b in range(NBUF):
            wait_gather(b)
            scale(b)
            start_scatter(b)
        for b in range(NBUF):
            wait_scatter(b)
            start_pf(j + NBUF + b, b)
            wait_pf(b)
            start_gather(j + NBUF + b, b)

    for b in range(NBUF):
        wait_gather(b)
        scale(b)
        start_scatter(b)
    for i, j in enumerate(range(NBUF * (steady + 1), NCHUNKS)):
        b = i
        wait_scatter(b)
        start_pf(j, b)
        wait_pf(b)
        start_gather(j, b)
    for i in range(NCHUNKS - NBUF * (steady + 1)):
        wait_gather(i)
        scale(i)
        start_scatter(i)
    for b in range(NBUF):
        wait_scatter(b)

    tbase = base_edge + NCHUNKS * CHUNK
    pltpu.sync_copy(eidx_hbm.at[pl.ds(tbase, TAIL)], sidx_t)
    pltpu.sync_copy(eidx_hbm.at[pl.ds(E_EDGES + tbase, TAIL)], didx_t)
    pltpu.sync_copy(attn_hbm.at[pl.ds(tbase, TAIL)], attn_t)
    trows = rows_refs[0].at[pl.ds(0, TAIL)]
    pltpu.sync_copy(hidden_hbm.at[sidx_t], trows)

    @pl.loop(0, TAIL)
    def _(e):
        a = plsc.load_gather(attn_t, [jnp.full((LANES,), e, jnp.int32)])
        for g in range(D // LANES):
            sl = pl.ds(g * LANES, LANES)
            rows_refs[0][e, sl] = rows_refs[0][e, sl] * a

    pltpu.sync_copy(trows, acc_ref.at[didx_t], add=True)

    plsc.subcore_barrier()
    pltpu.sync_copy(acc_ref.at[pl.ds(row0, ROWS_PER_TILE)],
                    out_hbm.at[c, pl.ds(row0, ROWS_PER_TILE)])


def _final_body(part_ref, w_ref, b_ref, out_ref):
    h = part_ref[0, :N_DST, :] + part_ref[1, :N_DST, :]
    rst = jax.lax.dot_general(
        h, w_ref[...],
        dimension_numbers=(((1,), (1,)), ((), ())),
        precision=lax.Precision.HIGHEST,
        preferred_element_type=jnp.float32)
    out_ref[...] = rst + b_ref[...][None, :]


@jax.jit
def kernel(hidden_feat, node_feat_src, node_feat_dst, sample_weights, q_probs,
           W_neigh, b_neigh, edge_index, deg_src, deg_dst):
    tabs = pl.pallas_call(
        _tables_body,
        out_shape=jax.ShapeDtypeStruct((4 * N_SRC,), jnp.float32),
    )(node_feat_src, node_feat_dst, sample_weights, q_probs, deg_src, deg_dst)

    eidx = edge_index.reshape(2 * E_EDGES)
    zeros_rows = jnp.zeros((ROWS_PER_TILE, D), jnp.float32)

    mesh = plsc.VectorSubcoreMesh(core_axis_name="c", subcore_axis_name="s")
    sc_params = pltpu.CompilerParams()
    if "needs_layout_passes" in pltpu.CompilerParams.__dataclass_fields__:
        sc_params = dataclasses.replace(sc_params, needs_layout_passes=False)

    attn_kernel = functools.partial(
        pl.kernel,
        compiler_params=sc_params,
        out_type=jax.ShapeDtypeStruct((E_EDGES,), jnp.float32),
        mesh=mesh,
        scratch_types=[
            pltpu.VMEM((N_SRC,), jnp.float32),
            pltpu.VMEM((N_SRC,), jnp.float32),
            pltpu.VMEM((N_DST,), jnp.float32),
            pltpu.VMEM((N_DST,), jnp.float32),
            pltpu.VMEM((EDGES_PER_TILE,), jnp.int32),
            pltpu.VMEM((EDGES_PER_TILE,), jnp.int32),
            pltpu.VMEM((EDGES_PER_TILE,), jnp.float32),
            pltpu.SemaphoreType.DMA,
        ],
    )(_attn_kernel_body)
    attn_all = attn_kernel(eidx, tabs)

    agg_kernel = functools.partial(
        pl.kernel,
        compiler_params=sc_params,
        out_type=jax.ShapeDtypeStruct((NUM_CORES, N_PAD, D), jnp.float32),
        mesh=mesh,
        scratch_types=[
            [pltpu.VMEM((CHUNK,), jnp.int32) for _ in range(NBUF)],
            [pltpu.VMEM((CHUNK,), jnp.int32) for _ in range(NBUF)],
            [pltpu.VMEM((CHUNK,), jnp.float32) for _ in range(NBUF)],
            [pltpu.VMEM((CHUNK, D), jnp.float32) for _ in range(NBUF)],
            pltpu.VMEM((TAIL,), jnp.int32),
            pltpu.VMEM((TAIL,), jnp.int32),
            pltpu.VMEM((TAIL,), jnp.float32),
            [pltpu.SemaphoreType.DMA for _ in range(NBUF)],
            [pltpu.SemaphoreType.DMA for _ in range(NBUF)],
            [pltpu.SemaphoreType.DMA for _ in range(NBUF)],
            pltpu.SemaphoreType.DMA,
            pltpu.VMEM_SHARED((N_PAD, D), jnp.float32),
        ],
    )(_agg_kernel_body)
    partials = agg_kernel(eidx, hidden_feat, attn_all, zeros_rows)

    rst = pl.pallas_call(
        _final_body,
        out_shape=jax.ShapeDtypeStruct((N_DST, OUT), jnp.float32),
    )(partials, W_neigh, b_neigh)
    return rst
